# Initial kernel scaffold; baseline (speedup 1.0000x reference)
import jax
import jax.numpy as jnp
from jax.experimental import pallas as pl


def kernel(node_features, edge_index, edge_attr, edge_weights, eW1, eb1, eW2, eb2, egW1, egb1, egW2, egb2, nW1, nb1, nW2, nb2, ngW1, ngb1, ngW2, ngb2, We, Wn):
    raise NotImplementedError("write your pallas kernel here")


# speedup vs baseline: 3.1219x; 3.1219x over previous
import functools

import jax
import jax.numpy as jnp
from jax import lax
from jax.experimental import pallas as pl
from jax.experimental.pallas import tpu as pltpu
from jax.experimental.pallas import tpu_sc as plsc

N_NODES = 10000
N_EDGES = 320000
D_NODE = 128
D_EDGE = 64
DEGREE = 64

NC = 2
NS = 16
NW = NC * NS
E_PER_W = N_EDGES // NW
CHUNK = 80
N_CHUNKS = E_PER_W // CHUNK
ROWS_PER_TILE = 624
TAIL_OFF = 15 * ROWS_PER_TILE
TAIL_ROWS = N_NODES - TAIL_OFF


@functools.lru_cache(maxsize=None)
def _get_sc_mesh():
    return plsc.VectorSubcoreMesh(core_axis_name="c", subcore_axis_name="s")


@functools.lru_cache(maxsize=None)
def _get_sc_gather():
    @functools.partial(
        pl.kernel,
        out_type=jax.ShapeDtypeStruct((N_EDGES, 2 * D_NODE), jnp.float32),
        mesh=_get_sc_mesh(),
        scratch_types=[
            pltpu.VMEM((CHUNK,), jnp.int32),
            pltpu.VMEM((CHUNK, D_NODE), jnp.float32),
            pltpu.VMEM((CHUNK,), jnp.int32),
            pltpu.VMEM((CHUNK, D_NODE), jnp.float32),
            pltpu.SemaphoreType.DMA,
            pltpu.SemaphoreType.DMA,
        ],
    )
    def _sc_gather(nf_hbm, src_hbm, dst_hbm, out_hbm,
                   idx_a, rows_a, idx_b, rows_b, sem_a, sem_b):
        wid = lax.axis_index("s") * NC + lax.axis_index("c")
        base = wid * E_PER_W

        def body(i, _):
            eoff = base + i * CHUNK
            pltpu.sync_copy(src_hbm.at[pl.ds(eoff, CHUNK)], idx_a)
            pltpu.sync_copy(dst_hbm.at[pl.ds(eoff, CHUNK)], idx_b)
            ca = pltpu.async_copy(nf_hbm.at[idx_a], rows_a, sem_a)
            cb = pltpu.async_copy(nf_hbm.at[idx_b], rows_b, sem_b)
            ca.wait()
            cb.wait()
            pltpu.sync_copy(rows_a, out_hbm.at[pl.ds(eoff, CHUNK), pl.ds(0, D_NODE)])
            pltpu.sync_copy(rows_b, out_hbm.at[pl.ds(eoff, CHUNK), pl.ds(D_NODE, D_NODE)])

        lax.fori_loop(0, N_CHUNKS, body, None)

    return _sc_gather


BE = 2560
N_BLOCKS = N_EDGES // BE


def _mlp_body(vij_ref, ea_ref, ew_ref,
              W1vij_ref, W1ea_e_ref, b1e_ref, W2e_ref, b2e_ref,
              W1ea_n_ref, b1n_ref, W2n_ref, b2n_ref, WeWn_ref,
              ea_new_ref, feats_ref):
    f32 = jnp.float32
    vij = vij_ref[...]
    ea = ea_ref[...]
    ew = ew_ref[...]

    pre1 = jnp.dot(vij, W1vij_ref[...], preferred_element_type=f32)
    ewp = jnp.dot(ew, WeWn_ref[...], preferred_element_type=f32)

    he = pre1[:, 0:128] + jnp.dot(ea, W1ea_e_ref[...], preferred_element_type=f32)
    he = he + b1e_ref[...]
    he = he * jax.nn.sigmoid(he)
    s2e = jnp.dot(he, W2e_ref[...], preferred_element_type=f32) + b2e_ref[...]
    ue = s2e[:, 0:64]
    ue = ue * jax.nn.sigmoid(ue)
    ge = jax.nn.sigmoid(s2e[:, 64:128])
    ea_new = ea + ue * ge * ewp[:, 0:64]
    ea_new_ref[...] = ea_new

    hn = pre1[:, 128:256] + jnp.dot(ea_new, W1ea_n_ref[...], preferred_element_type=f32)
    hn = hn + b1n_ref[...]
    hn = hn * jax.nn.sigmoid(hn)
    s2n = jnp.dot(hn, W2n_ref[...], preferred_element_type=f32) + b2n_ref[...]
    un = s2n[:, 0:128]
    un = un * jax.nn.sigmoid(un)
    gn = jax.nn.sigmoid(s2n[:, 128:256])
    feats_ref[...] = un * gn * ewp[:, 64:192]


def _run_mlp(vij, ea, ew, W1vij, W1ea_e, b1e, W2e, b2e, W1ea_n, b1n, W2n, b2n,
             WeWn):
    blk = lambda shape: pl.BlockSpec(shape, lambda i: (0,) * len(shape))
    ebs = lambda w: pl.BlockSpec((BE, w), lambda i: (i, 0))
    return pl.pallas_call(
        _mlp_body,
        grid=(N_BLOCKS,),
        in_specs=[
            ebs(256), ebs(64), ebs(64),
            blk((256, 256)), blk((64, 128)), blk((1, 128)), blk((128, 128)),
            blk((1, 128)), blk((64, 128)), blk((1, 128)), blk((128, 256)),
            blk((1, 256)), blk((64, 192)),
        ],
        out_specs=[ebs(64), ebs(128)],
        out_shape=[
            jax.ShapeDtypeStruct((N_EDGES, D_EDGE), jnp.float32),
            jax.ShapeDtypeStruct((N_EDGES, D_NODE), jnp.float32),
        ],
    )(vij, ea, ew, W1vij, W1ea_e, b1e, W2e, b2e, W1ea_n, b1n, W2n, b2n, WeWn)


@functools.lru_cache(maxsize=None)
def _get_sc_scatter():
    @functools.partial(
        pl.kernel,
        out_type=jax.ShapeDtypeStruct((NC, N_NODES, D_NODE), jnp.float32),
        mesh=_get_sc_mesh(),
        scratch_types=[
            pltpu.VMEM_SHARED((N_NODES, D_NODE), jnp.float32),
            pltpu.VMEM((CHUNK,), jnp.int32),
            pltpu.VMEM((CHUNK, D_NODE), jnp.float32),
        ],
    )
    def _sc_scatter(feats_hbm, src_hbm, nf_hbm, zeros_hbm, out_hbm,
                    acc, idx_v, rows_v):
        cid = lax.axis_index("c")
        sid = lax.axis_index("s")
        wid = sid * NC + cid
        base = wid * E_PER_W
        roff = sid * ROWS_PER_TILE

        @pl.when(cid == 0)
        def _():
            pltpu.sync_copy(nf_hbm.at[pl.ds(roff, ROWS_PER_TILE)],
                            acc.at[pl.ds(roff, ROWS_PER_TILE)])

            @pl.when(sid == NS - 1)
            def _():
                pltpu.sync_copy(nf_hbm.at[pl.ds(TAIL_OFF + ROWS_PER_TILE, TAIL_ROWS - ROWS_PER_TILE)],
                                acc.at[pl.ds(TAIL_OFF + ROWS_PER_TILE, TAIL_ROWS - ROWS_PER_TILE)])

        @pl.when(cid != 0)
        def _():
            pltpu.sync_copy(zeros_hbm.at[pl.ds(roff, ROWS_PER_TILE)],
                            acc.at[pl.ds(roff, ROWS_PER_TILE)])

            @pl.when(sid == NS - 1)
            def _():
                pltpu.sync_copy(zeros_hbm.at[pl.ds(TAIL_OFF + ROWS_PER_TILE, TAIL_ROWS - ROWS_PER_TILE)],
                                acc.at[pl.ds(TAIL_OFF + ROWS_PER_TILE, TAIL_ROWS - ROWS_PER_TILE)])

        plsc.subcore_barrier()

        def body(i, _):
            eoff = base + i * CHUNK
            pltpu.sync_copy(src_hbm.at[pl.ds(eoff, CHUNK)], idx_v)
            pltpu.sync_copy(feats_hbm.at[pl.ds(eoff, CHUNK)], rows_v)
            pltpu.sync_copy(rows_v, acc.at[idx_v], add=True)

        lax.fori_loop(0, N_CHUNKS, body, None)

        plsc.subcore_barrier()
        pltpu.sync_copy(acc.at[pl.ds(roff, ROWS_PER_TILE)],
                        out_hbm.at[cid, pl.ds(roff, ROWS_PER_TILE)])

        @pl.when(sid == NS - 1)
        def _():
            pltpu.sync_copy(acc.at[pl.ds(TAIL_OFF + ROWS_PER_TILE, TAIL_ROWS - ROWS_PER_TILE)],
                            out_hbm.at[cid, pl.ds(TAIL_OFF + ROWS_PER_TILE, TAIL_ROWS - ROWS_PER_TILE)])

    return _sc_scatter


def _combine_body(p_ref, out_ref):
    out_ref[...] = p_ref[0] + p_ref[1]


def _run_combine(parts):
    nb = 10
    rb = N_NODES // nb
    return pl.pallas_call(
        _combine_body,
        grid=(nb,),
        in_specs=[pl.BlockSpec((NC, rb, D_NODE), lambda i: (0, i, 0))],
        out_specs=pl.BlockSpec((rb, D_NODE), lambda i: (i, 0)),
        out_shape=jax.ShapeDtypeStruct((N_NODES, D_NODE), jnp.float32),
    )(parts)


def kernel(node_features, edge_index, edge_attr, edge_weights,
           eW1, eb1, eW2, eb2, egW1, egb1, egW2, egb2,
           nW1, nb1, nW2, nb2, ngW1, ngb1, ngW2, ngb2,
           We, Wn):
    src = edge_index[0].astype(jnp.int32)
    dst = edge_index[1].astype(jnp.int32)

    top = jnp.concatenate([eW1[0:128], egW1[0:128], nW1[0:128], ngW1[0:128]], axis=1)
    bot = jnp.concatenate([eW1[128:256], egW1[128:256], nW1[128:256], ngW1[128:256]], axis=1)
    W1vij = jnp.concatenate([top, bot], axis=0)
    W1ea_e = jnp.concatenate([eW1[256:320], egW1[256:320]], axis=1)
    W1ea_n = jnp.concatenate([nW1[256:320], ngW1[256:320]], axis=1)
    b1e = jnp.concatenate([eb1, egb1])[None, :]
    b1n = jnp.concatenate([nb1, ngb1])[None, :]
    z64 = jnp.zeros((64, 64), jnp.float32)
    W2e = jnp.block([[eW2, z64], [z64, egW2]])
    b2e = jnp.concatenate([eb2, egb2])[None, :]
    z64n = jnp.zeros((64, 128), jnp.float32)
    W2n = jnp.block([[nW2, z64n], [z64n, ngW2]])
    b2n = jnp.concatenate([nb2, ngb2])[None, :]
    WeWn = jnp.concatenate([We, Wn], axis=1)

    vij = _get_sc_gather()(node_features, src, dst)
    ea_new, feats = _run_mlp(vij, edge_attr, edge_weights,
                             W1vij, W1ea_e, b1e, W2e, b2e,
                             W1ea_n, b1n, W2n, b2n, WeWn)
    zeros = jnp.zeros((N_NODES, D_NODE), jnp.float32)
    parts = _get_sc_scatter()(feats, src, node_features, zeros)
    node_new = _run_combine(parts)
    return (node_new, ea_new)


# speedup vs baseline: 3.6550x; 1.1708x over previous
import functools

import jax
import jax.numpy as jnp
from jax import lax
from jax.experimental import pallas as pl
from jax.experimental.pallas import tpu as pltpu
from jax.experimental.pallas import tpu_sc as plsc

N_NODES = 10000
N_EDGES = 320000
D_NODE = 128
D_EDGE = 64
DEGREE = 64

NC = 2
NS = 16
NW = NC * NS
E_PER_W = N_EDGES // NW
CHUNK = 80
N_CHUNKS = E_PER_W // CHUNK
ROWS_PER_TILE = 624
TAIL_OFF = 15 * ROWS_PER_TILE
TAIL_ROWS = N_NODES - TAIL_OFF


@functools.lru_cache(maxsize=None)
def _get_sc_mesh():
    return plsc.VectorSubcoreMesh(core_axis_name="c", subcore_axis_name="s")


@functools.lru_cache(maxsize=None)
def _get_sc_gather():
    HW = D_NODE
    @functools.partial(
        pl.kernel,
        out_type=jax.ShapeDtypeStruct((N_EDGES, 2 * D_NODE), jnp.float32),
        mesh=_get_sc_mesh(),
        scratch_types=[
            pltpu.VMEM((N_CHUNKS, CHUNK), jnp.int32),
            pltpu.VMEM((N_CHUNKS, CHUNK), jnp.int32),
            pltpu.VMEM((CHUNK, 2 * D_NODE), jnp.float32),
            pltpu.VMEM((CHUNK, 2 * D_NODE), jnp.float32),
            pltpu.SemaphoreType.DMA,
            pltpu.SemaphoreType.DMA,
            pltpu.SemaphoreType.DMA,
        ],
    )
    def _sc_gather(nf_hbm, srcm_hbm, dstm_hbm, out_hbm,
                   idxs, idxd, ra, rb, sem_a, sem_b, sem_w):
        wid = lax.axis_index("s") * NC + lax.axis_index("c")
        base = wid * E_PER_W

        pltpu.sync_copy(srcm_hbm.at[wid], idxs)
        pltpu.sync_copy(dstm_hbm.at[wid], idxd)

        def rows(c):
            return pl.ds(base + c * CHUNK, CHUNK)

        def body(i, _):
            c0 = 2 * i
            c1 = 2 * i + 1
            g0s = pltpu.async_copy(nf_hbm.at[idxs.at[c0]], ra.at[:, pl.ds(0, HW)], sem_a)
            g0d = pltpu.async_copy(nf_hbm.at[idxd.at[c0]], ra.at[:, pl.ds(HW, HW)], sem_a)
            g1s = pltpu.async_copy(nf_hbm.at[idxs.at[c1]], rb.at[:, pl.ds(0, HW)], sem_b)
            g1d = pltpu.async_copy(nf_hbm.at[idxd.at[c1]], rb.at[:, pl.ds(HW, HW)], sem_b)
            g0s.wait()
            g0d.wait()
            w0 = pltpu.async_copy(ra, out_hbm.at[rows(c0)], sem_w)
            g1s.wait()
            g1d.wait()
            w1 = pltpu.async_copy(rb, out_hbm.at[rows(c1)], sem_w)
            w0.wait()
            w1.wait()

        lax.fori_loop(0, N_CHUNKS // 2, body, None)

        ct = N_CHUNKS - 1
        gts = pltpu.async_copy(nf_hbm.at[idxs.at[ct]], ra.at[:, pl.ds(0, HW)], sem_a)
        gtd = pltpu.async_copy(nf_hbm.at[idxd.at[ct]], ra.at[:, pl.ds(HW, HW)], sem_a)
        gts.wait()
        gtd.wait()
        wt = pltpu.async_copy(ra, out_hbm.at[rows(ct)], sem_w)
        wt.wait()

    return _sc_gather


BE = 2560
N_BLOCKS = N_EDGES // BE


def _mlp_body(vij_ref, ea_ref, ew_ref,
              Wv_lo_ref, Wv_hi_ref, W1ea_e_ref, b1e_ref, W2e_ref, b2e_ref,
              W1ea_n_ref, b1n_ref, W2n_ref, b2n_ref, WeWn_ref,
              ea_new_ref, feats_ref):
    f32 = jnp.float32
    bf = jnp.bfloat16
    v_lo = vij_ref[:, 0:D_NODE].astype(bf)
    v_hi = vij_ref[:, D_NODE:2 * D_NODE].astype(bf)
    ea = ea_ref[...]
    ea_bf = ea.astype(bf)
    ew = ew_ref[...].astype(bf)

    pre1 = (jnp.dot(v_lo, Wv_lo_ref[...], preferred_element_type=f32)
            + jnp.dot(v_hi, Wv_hi_ref[...], preferred_element_type=f32))
    ewp = jnp.dot(ew, WeWn_ref[...], preferred_element_type=f32)

    he = pre1[:, 0:128] + jnp.dot(ea_bf, W1ea_e_ref[...], preferred_element_type=f32)
    he = he + b1e_ref[...]
    he = he * jax.nn.sigmoid(he)
    s2e = jnp.dot(he.astype(bf), W2e_ref[...], preferred_element_type=f32) + b2e_ref[...]
    ue = s2e[:, 0:64]
    ue = ue * jax.nn.sigmoid(ue)
    ge = jax.nn.sigmoid(s2e[:, 64:128])
    ea_new = ea + ue * ge * ewp[:, 0:64]
    ea_new_ref[...] = ea_new

    hn = pre1[:, 128:256] + jnp.dot(ea_new.astype(bf), W1ea_n_ref[...],
                                    preferred_element_type=f32)
    hn = hn + b1n_ref[...]
    hn = hn * jax.nn.sigmoid(hn)
    s2n = jnp.dot(hn.astype(bf), W2n_ref[...], preferred_element_type=f32) + b2n_ref[...]
    un = s2n[:, 0:128]
    un = un * jax.nn.sigmoid(un)
    gn = jax.nn.sigmoid(s2n[:, 128:256])
    feats_ref[...] = un * gn * ewp[:, 64:192]


def _run_mlp(vij, ea, ew, Wv_lo, Wv_hi, W1ea_e, b1e, W2e, b2e, W1ea_n, b1n,
             W2n, b2n, WeWn):
    blk = lambda shape: pl.BlockSpec(shape, lambda i: (0,) * len(shape))
    ebs = lambda w: pl.BlockSpec((BE, w), lambda i: (i, 0))
    return pl.pallas_call(
        _mlp_body,
        grid=(N_BLOCKS,),
        in_specs=[
            ebs(256), ebs(64), ebs(64),
            blk((128, 256)), blk((128, 256)), blk((64, 128)), blk((1, 128)),
            blk((128, 128)), blk((1, 128)), blk((64, 128)), blk((1, 128)),
            blk((128, 256)), blk((1, 256)), blk((64, 192)),
        ],
        out_specs=[ebs(64), ebs(128)],
        out_shape=[
            jax.ShapeDtypeStruct((N_EDGES, D_EDGE), jnp.float32),
            jax.ShapeDtypeStruct((N_EDGES, D_NODE), jnp.float32),
        ],
    )(vij, ea, ew, Wv_lo, Wv_hi, W1ea_e, b1e, W2e, b2e, W1ea_n, b1n, W2n, b2n,
      WeWn)


@functools.lru_cache(maxsize=None)
def _get_sc_scatter():
    @functools.partial(
        pl.kernel,
        out_type=jax.ShapeDtypeStruct((NC, N_NODES, D_NODE), jnp.float32),
        mesh=_get_sc_mesh(),
        scratch_types=[
            pltpu.VMEM_SHARED((N_NODES, D_NODE), jnp.float32),
            pltpu.VMEM((N_CHUNKS, CHUNK), jnp.int32),
            pltpu.VMEM((CHUNK, D_NODE), jnp.float32),
            pltpu.VMEM((CHUNK, D_NODE), jnp.float32),
            pltpu.SemaphoreType.DMA,
            pltpu.SemaphoreType.DMA,
        ],
    )
    def _sc_scatter(feats_hbm, srcm_hbm, nf_hbm, zeros_hbm, out_hbm,
                    acc, idxs, rowa, rowb, sem_a, sem_b):
        cid = lax.axis_index("c")
        sid = lax.axis_index("s")
        wid = sid * NC + cid
        base = wid * E_PER_W
        roff = sid * ROWS_PER_TILE

        pltpu.sync_copy(srcm_hbm.at[wid], idxs)

        @pl.when(cid == 0)
        def _():
            pltpu.sync_copy(nf_hbm.at[pl.ds(roff, ROWS_PER_TILE)],
                            acc.at[pl.ds(roff, ROWS_PER_TILE)])

            @pl.when(sid == NS - 1)
            def _():
                pltpu.sync_copy(nf_hbm.at[pl.ds(TAIL_OFF + ROWS_PER_TILE, TAIL_ROWS - ROWS_PER_TILE)],
                                acc.at[pl.ds(TAIL_OFF + ROWS_PER_TILE, TAIL_ROWS - ROWS_PER_TILE)])

        @pl.when(cid != 0)
        def _():
            pltpu.sync_copy(zeros_hbm.at[pl.ds(roff, ROWS_PER_TILE)],
                            acc.at[pl.ds(roff, ROWS_PER_TILE)])

            @pl.when(sid == NS - 1)
            def _():
                pltpu.sync_copy(zeros_hbm.at[pl.ds(TAIL_OFF + ROWS_PER_TILE, TAIL_ROWS - ROWS_PER_TILE)],
                                acc.at[pl.ds(TAIL_OFF + ROWS_PER_TILE, TAIL_ROWS - ROWS_PER_TILE)])

        plsc.subcore_barrier()

        def rows(c):
            return pl.ds(base + c * CHUNK, CHUNK)

        def body(i, _):
            c0 = 2 * i
            c1 = 2 * i + 1
            f0 = pltpu.async_copy(feats_hbm.at[rows(c0)], rowa, sem_a)
            f1 = pltpu.async_copy(feats_hbm.at[rows(c1)], rowb, sem_b)
            f0.wait()
            pltpu.sync_copy(rowa, acc.at[idxs.at[c0]], add=True)
            f1.wait()
            pltpu.sync_copy(rowb, acc.at[idxs.at[c1]], add=True)

        lax.fori_loop(0, N_CHUNKS // 2, body, None)

        ct = N_CHUNKS - 1
        ft = pltpu.async_copy(feats_hbm.at[rows(ct)], rowa, sem_a)
        ft.wait()
        pltpu.sync_copy(rowa, acc.at[idxs.at[ct]], add=True)

        plsc.subcore_barrier()
        pltpu.sync_copy(acc.at[pl.ds(roff, ROWS_PER_TILE)],
                        out_hbm.at[cid, pl.ds(roff, ROWS_PER_TILE)])

        @pl.when(sid == NS - 1)
        def _():
            pltpu.sync_copy(acc.at[pl.ds(TAIL_OFF + ROWS_PER_TILE, TAIL_ROWS - ROWS_PER_TILE)],
                            out_hbm.at[cid, pl.ds(TAIL_OFF + ROWS_PER_TILE, TAIL_ROWS - ROWS_PER_TILE)])

    return _sc_scatter


def _combine_body(p_ref, out_ref):
    out_ref[...] = p_ref[0] + p_ref[1]


def _run_combine(parts):
    nb = 10
    rb = N_NODES // nb
    return pl.pallas_call(
        _combine_body,
        grid=(nb,),
        in_specs=[pl.BlockSpec((NC, rb, D_NODE), lambda i: (0, i, 0))],
        out_specs=pl.BlockSpec((rb, D_NODE), lambda i: (i, 0)),
        out_shape=jax.ShapeDtypeStruct((N_NODES, D_NODE), jnp.float32),
    )(parts)


def kernel(node_features, edge_index, edge_attr, edge_weights,
           eW1, eb1, eW2, eb2, egW1, egb1, egW2, egb2,
           nW1, nb1, nW2, nb2, ngW1, ngb1, ngW2, ngb2,
           We, Wn):
    bf = jnp.bfloat16
    src = edge_index[0].astype(jnp.int32)
    dst = edge_index[1].astype(jnp.int32)
    srcm = src.reshape(NW, N_CHUNKS, CHUNK)
    dstm = dst.reshape(NW, N_CHUNKS, CHUNK)

    top = jnp.concatenate([eW1[0:128], egW1[0:128], nW1[0:128], ngW1[0:128]], axis=1)
    bot = jnp.concatenate([eW1[128:256], egW1[128:256], nW1[128:256], ngW1[128:256]], axis=1)
    Wv_lo = top.astype(bf)
    Wv_hi = bot.astype(bf)
    W1ea_e = jnp.concatenate([eW1[256:320], egW1[256:320]], axis=1).astype(bf)
    W1ea_n = jnp.concatenate([nW1[256:320], ngW1[256:320]], axis=1).astype(bf)
    b1e = jnp.concatenate([eb1, egb1])[None, :]
    b1n = jnp.concatenate([nb1, ngb1])[None, :]
    z64 = jnp.zeros((64, 64), jnp.float32)
    W2e = jnp.block([[eW2, z64], [z64, egW2]]).astype(bf)
    b2e = jnp.concatenate([eb2, egb2])[None, :]
    z64n = jnp.zeros((64, 128), jnp.float32)
    W2n = jnp.block([[nW2, z64n], [z64n, ngW2]]).astype(bf)
    b2n = jnp.concatenate([nb2, ngb2])[None, :]
    WeWn = jnp.concatenate([We, Wn], axis=1).astype(bf)

    vij = _get_sc_gather()(node_features, srcm, dstm)
    ea_new, feats = _run_mlp(vij, edge_attr, edge_weights,
                             Wv_lo, Wv_hi, W1ea_e, b1e, W2e, b2e,
                             W1ea_n, b1n, W2n, b2n, WeWn)
    zeros = jnp.zeros((N_NODES, D_NODE), jnp.float32)
    parts = _get_sc_scatter()(feats, srcm, node_features, zeros)
    node_new = _run_combine(parts)
    return (node_new, ea_new)
